# Initial kernel scaffold; baseline (speedup 1.0000x reference)
import jax
import jax.numpy as jnp
from jax.experimental import pallas as pl


def kernel(x, adj_t, W1, b1, W2, b2, W3, b3, g1, be1, g2, be2):
    raise NotImplementedError("write your pallas kernel here")


# speedup vs baseline: 7.1081x; 7.1081x over previous
import functools

import jax
import jax.numpy as jnp
from jax import lax
from jax.experimental import pallas as pl
from jax.experimental.pallas import tpu as pltpu
from jax.experimental.pallas import tpu_sc as plsc

N = 10000
E = 320000
D_IN = 128
D_HID = 128
D_OUT = 40
DPAD = 128
EPS = 1e-5

NC = 2
NS = 16
NW = NC * NS
NR = 10240
RPT = NR // NS
CB = 128
CBN = RPT // CB
K = 128
EWP = 10240
NCH = EWP // K
EP = NW * EWP
DPT = NR // NS

_MESH = plsc.VectorSubcoreMesh(core_axis_name="c", subcore_axis_name="s")


@functools.partial(
    pl.kernel,
    out_type=jax.ShapeDtypeStruct((NC, NR), jnp.float32),
    mesh=_MESH,
    scratch_types=[
        pltpu.VMEM((K,), jnp.int32),
        pltpu.VMEM((K,), jnp.float32),
        pltpu.VMEM((DPT,), jnp.float32),
        pltpu.VMEM_SHARED((NR,), jnp.float32),
    ],
)
def _sc_degree(dst_hbm, out_hbm, didx, ones_v, zbuf, acc):
    c = lax.axis_index("c")
    s = lax.axis_index("s")
    wid = s * NC + c

    for i in range(K // 16):
        ones_v[pl.ds(i * 16, 16)] = jnp.ones((16,), jnp.float32)

    def zfill(i, carry):
        zbuf[pl.ds(i * 16, 16)] = jnp.zeros((16,), jnp.float32)
        return carry

    lax.fori_loop(0, DPT // 16, zfill, 0)
    pltpu.sync_copy(zbuf, acc.at[pl.ds(s * DPT, DPT)])
    plsc.subcore_barrier()

    eb = wid * EWP

    def ebody(i, carry):
        pltpu.sync_copy(dst_hbm.at[pl.ds(eb + i * K, K)], didx)
        pltpu.sync_copy(ones_v, acc.at[didx], add=True)
        return carry

    lax.fori_loop(0, NCH, ebody, 0)
    plsc.subcore_barrier()

    pltpu.sync_copy(acc.at[pl.ds(s * DPT, DPT)], zbuf)
    pltpu.sync_copy(zbuf, out_hbm.at[c, pl.ds(s * DPT, DPT)])


def _make_sc_agg(D):
    @functools.partial(
        pl.kernel,
        out_type=jax.ShapeDtypeStruct((NC, NR, D), jnp.float32),
        mesh=_MESH,
        scratch_types=[
            pltpu.VMEM((K,), jnp.int32),
            pltpu.VMEM((K,), jnp.int32),
            pltpu.VMEM((K, D), jnp.float32),
            pltpu.VMEM((CB, D), jnp.float32),
            pltpu.VMEM_SHARED((NR, D), jnp.float32),
            pltpu.SemaphoreType.DMA,
        ],
    )
    def _sc_agg(y_hbm, src_hbm, dst_hbm, out_hbm, sidx, didx, rows, cbuf, acc,
                sem):
        c = lax.axis_index("c")
        s = lax.axis_index("s")
        wid = s * NC + c
        r0 = s * RPT

        @pl.when(c == 0)
        def _():
            def ibody(j, carry):
                pltpu.sync_copy(y_hbm.at[pl.ds(r0 + j * CB, CB)], cbuf)
                pltpu.sync_copy(cbuf, acc.at[pl.ds(r0 + j * CB, CB)])
                return carry

            lax.fori_loop(0, CBN, ibody, 0)

        @pl.when(c == 1)
        def _():
            def zfill(i, carry):
                j = i // (D // 16)
                col = (i % (D // 16)) * 16
                cbuf[j, pl.ds(col, 16)] = jnp.zeros((16,), jnp.float32)
                return carry

            lax.fori_loop(0, CB * D // 16, zfill, 0)

            def zbody(j, carry):
                pltpu.sync_copy(cbuf, acc.at[pl.ds(r0 + j * CB, CB)])
                return carry

            lax.fori_loop(0, CBN, zbody, 0)

        plsc.subcore_barrier()

        eb = wid * EWP

        def ebody(i, carry):
            base = eb + i * K
            pltpu.sync_copy(src_hbm.at[pl.ds(base, K)], sidx)
            pltpu.sync_copy(dst_hbm.at[pl.ds(base, K)], didx)
            pltpu.async_copy(y_hbm.at[sidx], rows, sem).wait()
            pltpu.sync_copy(rows, acc.at[didx], add=True)
            return carry

        lax.fori_loop(0, NCH, ebody, 0)
        plsc.subcore_barrier()

        def wbody(j, carry):
            pltpu.sync_copy(acc.at[pl.ds(r0 + j * CB, CB)], cbuf)
            pltpu.sync_copy(cbuf, out_hbm.at[c, pl.ds(r0 + j * CB, CB)])
            return carry

        lax.fori_loop(0, CBN, wbody, 0)

    return _sc_agg


_sc_agg_hid = _make_sc_agg(D_HID)
_sc_agg_out = _sc_agg_hid


def _tc_pre_body(deg_ref, x_ref, w_ref, y_ref):
    dinv = lax.rsqrt(deg_ref[...])
    xw = jnp.dot(x_ref[...], w_ref[...], preferred_element_type=jnp.float32)
    y_ref[:N, :] = xw * dinv
    y_ref[N:, :] = jnp.zeros((NR - N, w_ref.shape[1]), jnp.float32)


def _tc_pre(deg_col, x, W):
    return pl.pallas_call(
        _tc_pre_body,
        out_shape=jax.ShapeDtypeStruct((NR, W.shape[1]), jnp.float32),
    )(deg_col, x, W)


def _tc_mid_body(agg_ref, deg_ref, b_ref, g_ref, be_ref, w_ref, y_ref):
    dinv = lax.rsqrt(deg_ref[...])
    h = (agg_ref[0, :N, :] + agg_ref[1, :N, :]) * dinv + b_ref[...][None, :]
    mean = jnp.mean(h, axis=0, keepdims=True)
    cen = h - mean
    var = jnp.mean(cen * cen, axis=0, keepdims=True)
    hn = cen * lax.rsqrt(var + EPS) * g_ref[...][None, :] + be_ref[...][None, :]
    h = jnp.maximum(hn, 0.0)
    y_ref[:N, :] = jnp.dot(h, w_ref[...],
                           preferred_element_type=jnp.float32) * dinv
    y_ref[N:, :] = jnp.zeros((NR - N, w_ref.shape[1]), jnp.float32)


def _tc_mid(agg, deg_col, b, g, be, Wn):
    return pl.pallas_call(
        _tc_mid_body,
        out_shape=jax.ShapeDtypeStruct((NR, Wn.shape[1]), jnp.float32),
    )(agg, deg_col, b, g, be, Wn)


def _tc_post_body(agg_ref, deg_ref, b_ref, out_ref):
    dinv = lax.rsqrt(deg_ref[...])
    h48 = (agg_ref[0, :N, :] + agg_ref[1, :N, :]) * dinv
    h = h48[:, :D_OUT] + b_ref[...][None, :]
    m = jnp.max(h, axis=1, keepdims=True)
    e = jnp.exp(h - m)
    ssum = jnp.sum(e, axis=1, keepdims=True)
    out_ref[...] = h - m - jnp.log(ssum)


def _tc_post(agg, deg_col, b):
    return pl.pallas_call(
        _tc_post_body,
        out_shape=jax.ShapeDtypeStruct((N, D_OUT), jnp.float32),
    )(agg, deg_col, b)


def _pad_edges(e):
    return jnp.pad(e.reshape(NW, E // NW), ((0, 0), (0, EWP - E // NW)),
                   constant_values=N).reshape(EP)


def kernel(x, adj_t, W1, b1, W2, b2, W3, b3, g1, be1, g2, be2):
    src = _pad_edges(adj_t[0])
    dst = _pad_edges(adj_t[1])

    deg2 = _sc_degree(dst)
    deg_col = (deg2[0, :N] + deg2[1, :N] + 1.0)[:, None]

    y1 = _tc_pre(deg_col, x, W1)
    a1 = _sc_agg_hid(y1, src, dst)
    y2 = _tc_mid(a1, deg_col, b1, g1, be1, W2)
    a2 = _sc_agg_hid(y2, src, dst)
    W3p = jnp.pad(W3, ((0, 0), (0, DPAD - D_OUT)))
    y3 = _tc_mid(a2, deg_col, b2, g2, be2, W3p)
    a3 = _sc_agg_out(y3, src, dst)
    return _tc_post(a3, deg_col, b3)


# speedup vs baseline: 8.9878x; 1.2644x over previous
import functools

import jax
import jax.numpy as jnp
from jax import lax
from jax.experimental import pallas as pl
from jax.experimental.pallas import tpu as pltpu
from jax.experimental.pallas import tpu_sc as plsc

N = 10000
E = 320000
D_IN = 128
D_HID = 128
D_OUT = 40
DPAD = 128
EPS = 1e-5

NC = 2
NS = 16
NW = NC * NS
NR = 10240
RPT = NR // NS
CB = 128
CBN = RPT // CB
K = 128
EWP = 10240
NCH = EWP // K
EP = NW * EWP
DPT = NR // NS

_MESH = plsc.VectorSubcoreMesh(core_axis_name="c", subcore_axis_name="s")


@functools.partial(
    pl.kernel,
    out_type=jax.ShapeDtypeStruct((NC, NR), jnp.float32),
    mesh=_MESH,
    scratch_types=[
        pltpu.VMEM((NCH, K), jnp.int32),
        pltpu.VMEM((K,), jnp.float32),
        pltpu.VMEM((DPT,), jnp.float32),
        pltpu.VMEM_SHARED((NR,), jnp.float32),
    ],
)
def _sc_degree(dst_hbm, out_hbm, didx2, ones_v, zbuf, acc):
    c = lax.axis_index("c")
    s = lax.axis_index("s")
    wid = s * NC + c

    pltpu.sync_copy(dst_hbm.at[wid], didx2)
    for i in range(K // 16):
        ones_v[pl.ds(i * 16, 16)] = jnp.ones((16,), jnp.float32)

    def zfill(i, carry):
        zbuf[pl.ds(i * 16, 16)] = jnp.zeros((16,), jnp.float32)
        return carry

    lax.fori_loop(0, DPT // 16, zfill, 0)
    pltpu.sync_copy(zbuf, acc.at[pl.ds(s * DPT, DPT)])
    plsc.subcore_barrier()

    def ebody(i, carry):
        pltpu.sync_copy(ones_v, acc.at[didx2.at[i]], add=True)
        return carry

    lax.fori_loop(0, NCH, ebody, 0)
    plsc.subcore_barrier()

    pltpu.sync_copy(acc.at[pl.ds(s * DPT, DPT)], zbuf)
    pltpu.sync_copy(zbuf, out_hbm.at[c, pl.ds(s * DPT, DPT)])


NRB = 2
NIB = 4


def _make_sc_agg(D):
    @functools.partial(
        pl.kernel,
        out_type=jax.ShapeDtypeStruct((NC, NR, D), jnp.float32),
        mesh=_MESH,
        scratch_types=(
            [
                pltpu.VMEM((NIB, K), jnp.int32),
                pltpu.VMEM((NIB, K), jnp.int32),
            ]
            + [pltpu.VMEM((K, D), jnp.float32) for _ in range(NRB)]
            + [pltpu.VMEM_SHARED((NR, D), jnp.float32)]
            + [pltpu.SemaphoreType.DMA for _ in range(2 * NRB + NIB)]
        ),
    )
    def _sc_agg(y_hbm, src_hbm, dst_hbm, out_hbm, idxs, idxd,
                rb0, rb1, acc, g0, g1, s0, s1, i0, i1, i2, i3):
        rows = [rb0, rb1]
        semg = [g0, g1]
        sems = [s0, s1]
        semi = [i0, i1, i2, i3]
        c = lax.axis_index("c")
        s = lax.axis_index("s")
        wid = s * NC + c
        r0 = s * RPT
        cbuf = rb0

        def fetch_idx(ci, t):
            pltpu.async_copy(src_hbm.at[wid, ci], idxs.at[t], semi[t])
            pltpu.async_copy(dst_hbm.at[wid, ci], idxd.at[t], semi[t])

        def wait_idx(t):
            pltpu.make_async_copy(src_hbm.at[0, 0], idxs.at[t],
                                  semi[t]).wait()
            pltpu.make_async_copy(dst_hbm.at[0, 0], idxd.at[t],
                                  semi[t]).wait()

        def start_gather(ci_t, b):
            pltpu.async_copy(y_hbm.at[idxs.at[ci_t]], rows[b], semg[b])

        def wait_gather(b):
            pltpu.make_async_copy(y_hbm.at[pl.ds(0, K)], rows[b],
                                  semg[b]).wait()

        def start_scatter(ci_t, b):
            pltpu.async_copy(rows[b], acc.at[idxd.at[ci_t]], sems[b],
                             add=True)

        def wait_scatter(b):
            pltpu.make_async_copy(rows[b], acc.at[pl.ds(0, K)],
                                  sems[b]).wait()

        for t in range(NIB):
            fetch_idx(t, t)

        @pl.when(c == 0)
        def _():
            def ibody(j, carry):
                pltpu.sync_copy(y_hbm.at[pl.ds(r0 + j * CB, CB)], cbuf)
                pltpu.sync_copy(cbuf, acc.at[pl.ds(r0 + j * CB, CB)])
                return carry

            lax.fori_loop(0, CBN, ibody, 0)

        @pl.when(c == 1)
        def _():
            def zfill(i, carry):
                j = i // (D // 16)
                col = (i % (D // 16)) * 16
                cbuf[j, pl.ds(col, 16)] = jnp.zeros((16,), jnp.float32)
                return carry

            lax.fori_loop(0, CB * D // 16, zfill, 0)

            def zbody(j, carry):
                pltpu.sync_copy(cbuf, acc.at[pl.ds(r0 + j * CB, CB)])
                return carry

            lax.fori_loop(0, CBN, zbody, 0)

        plsc.subcore_barrier()

        wait_idx(0)
        start_gather(0, 0)
        wait_idx(1)
        start_gather(1, 1)
        wait_gather(0)
        start_scatter(0, 0)
        wait_gather(1)
        start_scatter(1, 1)
        wait_scatter(0)
        fetch_idx(4, 0)
        wait_idx(2)
        start_gather(2, 0)
        wait_scatter(1)
        fetch_idx(5, 1)
        wait_idx(3)
        start_gather(3, 1)
        wait_gather(0)
        start_scatter(2, 0)
        wait_gather(1)
        start_scatter(3, 1)

        def rbody(j, carry):
            for b4 in range(4):
                ci = j * 4 + b4
                b = b4 % 2
                wait_scatter(b)
                cn = jnp.minimum(ci + 2, NCH - 1)
                fetch_idx(cn, (b4 + 2) % 4)
                wait_idx(b4)
                start_gather(b4, b)
                wait_gather(b)
                start_scatter(b4, b)
            return carry

        lax.fori_loop(1, NCH // 4, rbody, 0)

        wait_scatter(0)
        wait_scatter(1)
        wait_idx(0)
        wait_idx(1)
        plsc.subcore_barrier()

        def wbody(j, carry):
            pltpu.sync_copy(acc.at[pl.ds(r0 + j * CB, CB)], cbuf)
            pltpu.sync_copy(cbuf, out_hbm.at[c, pl.ds(r0 + j * CB, CB)])
            return carry

        lax.fori_loop(0, CBN, wbody, 0)

    return _sc_agg


_sc_agg_hid = _make_sc_agg(D_HID)
_sc_agg_out = _sc_agg_hid


def _tc_pre_body(deg_ref, x_ref, w_ref, y_ref):
    dinv = lax.rsqrt(deg_ref[...])
    xw = jnp.dot(x_ref[...], w_ref[...], preferred_element_type=jnp.float32)
    y_ref[:N, :] = xw * dinv
    y_ref[N:, :] = jnp.zeros((NR - N, w_ref.shape[1]), jnp.float32)


def _tc_pre(deg_col, x, W):
    return pl.pallas_call(
        _tc_pre_body,
        out_shape=jax.ShapeDtypeStruct((NR, W.shape[1]), jnp.float32),
    )(deg_col, x, W)


def _tc_mid_body(agg_ref, deg_ref, b_ref, g_ref, be_ref, w_ref, y_ref):
    dinv = lax.rsqrt(deg_ref[...])
    h = (agg_ref[0, :N, :] + agg_ref[1, :N, :]) * dinv + b_ref[...][None, :]
    mean = jnp.mean(h, axis=0, keepdims=True)
    cen = h - mean
    var = jnp.mean(cen * cen, axis=0, keepdims=True)
    hn = cen * lax.rsqrt(var + EPS) * g_ref[...][None, :] + be_ref[...][None, :]
    h = jnp.maximum(hn, 0.0)
    y_ref[:N, :] = jnp.dot(h, w_ref[...],
                           preferred_element_type=jnp.float32) * dinv
    y_ref[N:, :] = jnp.zeros((NR - N, w_ref.shape[1]), jnp.float32)


def _tc_mid(agg, deg_col, b, g, be, Wn):
    return pl.pallas_call(
        _tc_mid_body,
        out_shape=jax.ShapeDtypeStruct((NR, Wn.shape[1]), jnp.float32),
    )(agg, deg_col, b, g, be, Wn)


def _tc_post_body(agg_ref, deg_ref, b_ref, out_ref):
    dinv = lax.rsqrt(deg_ref[...])
    h48 = (agg_ref[0, :N, :] + agg_ref[1, :N, :]) * dinv
    h = h48[:, :D_OUT] + b_ref[...][None, :]
    m = jnp.max(h, axis=1, keepdims=True)
    e = jnp.exp(h - m)
    ssum = jnp.sum(e, axis=1, keepdims=True)
    out_ref[...] = h - m - jnp.log(ssum)


def _tc_post(agg, deg_col, b):
    return pl.pallas_call(
        _tc_post_body,
        out_shape=jax.ShapeDtypeStruct((N, D_OUT), jnp.float32),
    )(agg, deg_col, b)


def _pad_edges(e):
    return jnp.pad(e.reshape(NW, E // NW), ((0, 0), (0, EWP - E // NW)),
                   constant_values=N).reshape(NW, NCH, K)


def kernel(x, adj_t, W1, b1, W2, b2, W3, b3, g1, be1, g2, be2):
    src = _pad_edges(adj_t[0])
    dst = _pad_edges(adj_t[1])

    deg2 = _sc_degree(dst)
    deg_col = (deg2[0, :N] + deg2[1, :N] + 1.0)[:, None]

    y1 = _tc_pre(deg_col, x, W1)
    a1 = _sc_agg_hid(y1, src, dst)
    y2 = _tc_mid(a1, deg_col, b1, g1, be1, W2)
    a2 = _sc_agg_hid(y2, src, dst)
    W3p = jnp.pad(W3, ((0, 0), (0, DPAD - D_OUT)))
    y3 = _tc_mid(a2, deg_col, b2, g2, be2, W3p)
    a3 = _sc_agg_out(y3, src, dst)
    return _tc_post(a3, deg_col, b3)


# speedup vs baseline: 9.3753x; 1.0431x over previous
import functools

import jax
import jax.numpy as jnp
from jax import lax
from jax.experimental import pallas as pl
from jax.experimental.pallas import tpu as pltpu
from jax.experimental.pallas import tpu_sc as plsc

N = 10000
E = 320000
D_IN = 128
D_HID = 128
D_OUT = 40
DPAD = 128
EPS = 1e-5

NC = 2
NS = 16
NW = NC * NS
NR = 10240
RPT = NR // NS
CB = 80
CBN = RPT // CB
K = 128
NCH = 80
EWP = NCH * K
EP = NW * EWP
DPT = NR // NS

_MESH = plsc.VectorSubcoreMesh(core_axis_name="c", subcore_axis_name="s")


@functools.partial(
    pl.kernel,
    out_type=jax.ShapeDtypeStruct((NC, NR), jnp.float32),
    mesh=_MESH,
    scratch_types=[
        pltpu.VMEM((NCH, K), jnp.int32),
        pltpu.VMEM((128,), jnp.float32),
        pltpu.VMEM((DPT,), jnp.float32),
        pltpu.VMEM_SHARED((NR,), jnp.float32),
    ],
)
def _sc_degree(dst_hbm, out_hbm, didx2, ones_v, zbuf, acc):
    c = lax.axis_index("c")
    s = lax.axis_index("s")
    wid = s * NC + c

    pltpu.sync_copy(dst_hbm.at[wid], didx2)
    for i in range(128 // 16):
        ones_v[pl.ds(i * 16, 16)] = jnp.ones((16,), jnp.float32)

    def zfill(i, carry):
        zbuf[pl.ds(i * 16, 16)] = jnp.zeros((16,), jnp.float32)
        return carry

    lax.fori_loop(0, DPT // 16, zfill, 0)
    pltpu.sync_copy(zbuf, acc.at[pl.ds(s * DPT, DPT)])
    plsc.subcore_barrier()

    def ebody(i, carry):
        pltpu.sync_copy(ones_v.at[pl.ds(0, K)], acc.at[didx2.at[i]],
                        add=True)
        return carry

    lax.fori_loop(0, NCH, ebody, 0)
    plsc.subcore_barrier()

    pltpu.sync_copy(acc.at[pl.ds(s * DPT, DPT)], zbuf)
    pltpu.sync_copy(zbuf, out_hbm.at[c, pl.ds(s * DPT, DPT)])


NRB = 2
NIB = 4
PD = NIB - NRB
G = 1


def _make_sc_agg(D):
    @functools.partial(
        pl.kernel,
        out_type=jax.ShapeDtypeStruct((NC, NR, D), jnp.float32),
        mesh=_MESH,
        scratch_types=(
            [
                pltpu.VMEM((NIB, K), jnp.int32),
                pltpu.VMEM((NIB, K), jnp.int32),
            ]
            + [pltpu.VMEM((K, D), jnp.float32) for _ in range(NRB)]
            + [pltpu.VMEM_SHARED((NR, D), jnp.float32)]
            + [pltpu.SemaphoreType.DMA for _ in range(2 * NRB + NIB)]
        ),
    )
    def _sc_agg(y_hbm, src_hbm, dst_hbm, out_hbm, *scr):
        idxs, idxd = scr[0], scr[1]
        rows = list(scr[2:2 + NRB])
        acc = scr[2 + NRB]
        semg = list(scr[3 + NRB:3 + 2 * NRB])
        sems = list(scr[3 + 2 * NRB:3 + 3 * NRB])
        semi = list(scr[3 + 3 * NRB:3 + 3 * NRB + NIB])
        c = lax.axis_index("c")
        s = lax.axis_index("s")
        wid = s * NC + c
        r0 = s * RPT

        def fetch_idx(ci, t):
            pltpu.async_copy(src_hbm.at[wid, ci], idxs.at[t], semi[t])
            pltpu.async_copy(dst_hbm.at[wid, ci], idxd.at[t], semi[t])

        def wait_idx(t):
            pltpu.make_async_copy(src_hbm.at[0, 0], idxs.at[t],
                                  semi[t]).wait()
            pltpu.make_async_copy(dst_hbm.at[0, 0], idxd.at[t],
                                  semi[t]).wait()

        def start_gather(t, b):
            pltpu.async_copy(y_hbm.at[idxs.at[t]], rows[b], semg[b])

        def wait_gather(b):
            pltpu.make_async_copy(y_hbm.at[pl.ds(0, K)], rows[b],
                                  semg[b]).wait()

        def start_scatter(t, b):
            pltpu.async_copy(rows[b], acc.at[idxd.at[t]], sems[b],
                             add=True)

        def wait_scatter(b):
            pltpu.make_async_copy(rows[b], acc.at[pl.ds(0, K)],
                                  sems[b]).wait()

        for t in range(PD):
            fetch_idx(t, t)

        cb = rows[0].at[pl.ds(0, CB)]

        @pl.when(c == 0)
        def _():
            def ibody(j, carry):
                pltpu.sync_copy(y_hbm.at[pl.ds(r0 + j * CB, CB)], cb)
                pltpu.sync_copy(cb, acc.at[pl.ds(r0 + j * CB, CB)])
                return carry

            lax.fori_loop(0, CBN, ibody, 0)

        @pl.when(c == 1)
        def _():
            def zfill(i, carry):
                j = i // (D // 16)
                col = (i % (D // 16)) * 16
                rows[0][j, pl.ds(col, 16)] = jnp.zeros((16,), jnp.float32)
                return carry

            lax.fori_loop(0, CB * D // 16, zfill, 0)

            def zbody(j, carry):
                pltpu.sync_copy(cb, acc.at[pl.ds(r0 + j * CB, CB)])
                return carry

            lax.fori_loop(0, CBN, zbody, 0)

        plsc.subcore_barrier()

        for ci in range(NIB):
            b = ci % NRB
            if ci >= NRB:
                wait_scatter(b)
            fetch_idx(ci + PD, (ci + PD) % NIB)
            wait_idx(ci % NIB)
            start_gather(ci % NIB, b)
            if ci >= G:
                pg = ci - G
                wait_gather(pg % NRB)
                start_scatter(pg % NIB, pg % NRB)

        def rbody(r, carry):
            for k in range(NIB):
                ci = r * NIB + k
                b = k % NRB
                wait_scatter(b)
                cn = jnp.minimum(ci + PD, NCH - 1)
                fetch_idx(cn, (k + PD) % NIB)
                wait_idx(k)
                start_gather(k, b)
                pg = (k - G) % NIB
                wait_gather(pg % NRB)
                start_scatter(pg, pg % NRB)
            return carry

        lax.fori_loop(1, NCH // NIB, rbody, 0)

        for gg in range(G):
            pg = (NCH - G + gg) % NIB
            wait_gather(pg % NRB)
            start_scatter(pg, pg % NRB)
        for b in range(NRB):
            wait_scatter(b)
        for t in range(PD):
            wait_idx(t)
        plsc.subcore_barrier()

        def wbody(j, carry):
            pltpu.sync_copy(acc.at[pl.ds(r0 + j * CB, CB)], cb)
            pltpu.sync_copy(cb, out_hbm.at[c, pl.ds(r0 + j * CB, CB)])
            return carry

        lax.fori_loop(0, CBN, wbody, 0)

    return _sc_agg


_sc_agg_hid = _make_sc_agg(D_HID)
_sc_agg_out = _sc_agg_hid


def _tc_pre_body(deg_ref, x_ref, w_ref, y_ref):
    dinv = lax.rsqrt(deg_ref[...])
    xw = jnp.dot(x_ref[...], w_ref[...], preferred_element_type=jnp.float32)
    y_ref[:N, :] = xw * dinv
    y_ref[N:, :] = jnp.zeros((NR - N, w_ref.shape[1]), jnp.float32)


def _tc_pre(deg_col, x, W):
    return pl.pallas_call(
        _tc_pre_body,
        out_shape=jax.ShapeDtypeStruct((NR, W.shape[1]), jnp.float32),
    )(deg_col, x, W)


def _tc_mid_body(agg_ref, deg_ref, b_ref, g_ref, be_ref, w_ref, y_ref):
    dinv = lax.rsqrt(deg_ref[...])
    h = (agg_ref[0, :N, :] + agg_ref[1, :N, :]) * dinv + b_ref[...][None, :]
    mean = jnp.mean(h, axis=0, keepdims=True)
    cen = h - mean
    var = jnp.mean(cen * cen, axis=0, keepdims=True)
    hn = cen * lax.rsqrt(var + EPS) * g_ref[...][None, :] + be_ref[...][None, :]
    h = jnp.maximum(hn, 0.0)
    y_ref[:N, :] = jnp.dot(h, w_ref[...],
                           preferred_element_type=jnp.float32) * dinv
    y_ref[N:, :] = jnp.zeros((NR - N, w_ref.shape[1]), jnp.float32)


def _tc_mid(agg, deg_col, b, g, be, Wn):
    return pl.pallas_call(
        _tc_mid_body,
        out_shape=jax.ShapeDtypeStruct((NR, Wn.shape[1]), jnp.float32),
    )(agg, deg_col, b, g, be, Wn)


def _tc_post_body(agg_ref, deg_ref, b_ref, out_ref):
    dinv = lax.rsqrt(deg_ref[...])
    h48 = (agg_ref[0, :N, :] + agg_ref[1, :N, :]) * dinv
    h = h48[:, :D_OUT] + b_ref[...][None, :]
    m = jnp.max(h, axis=1, keepdims=True)
    e = jnp.exp(h - m)
    ssum = jnp.sum(e, axis=1, keepdims=True)
    out_ref[...] = h - m - jnp.log(ssum)


def _tc_post(agg, deg_col, b):
    return pl.pallas_call(
        _tc_post_body,
        out_shape=jax.ShapeDtypeStruct((N, D_OUT), jnp.float32),
    )(agg, deg_col, b)


def _pad_edges(e):
    return jnp.pad(e.reshape(NW, E // NW), ((0, 0), (0, EWP - E // NW)),
                   constant_values=N).reshape(NW, NCH, K)


def kernel(x, adj_t, W1, b1, W2, b2, W3, b3, g1, be1, g2, be2):
    src = _pad_edges(adj_t[0])
    dst = _pad_edges(adj_t[1])

    deg2 = _sc_degree(dst)
    deg_col = (deg2[0, :N] + deg2[1, :N] + 1.0)[:, None]

    y1 = _tc_pre(deg_col, x, W1)
    a1 = _sc_agg_hid(y1, src, dst)
    y2 = _tc_mid(a1, deg_col, b1, g1, be1, W2)
    a2 = _sc_agg_hid(y2, src, dst)
    W3p = jnp.pad(W3, ((0, 0), (0, DPAD - D_OUT)))
    y3 = _tc_mid(a2, deg_col, b2, g2, be2, W3p)
    a3 = _sc_agg_out(y3, src, dst)
    return _tc_post(a3, deg_col, b3)
